# baseline (speedup 1.0000x reference)
import functools

import jax
import jax.numpy as jnp
from jax import lax
from jax.experimental import pallas as pl
from jax.experimental.pallas import tpu as pltpu
from jax.experimental.pallas import tpu_sc as plsc

N_NODES = 10000
D_FEAT = 128
N_EDGES = 160000
EPS = 1e-15
TABLE_SIZE = N_NODES * N_NODES
TABLE_PAD = TABLE_SIZE + 32 * 16

NUM_CORES = 2
NUM_SUBCORES = 16
NW = NUM_CORES * NUM_SUBCORES
ROWS_PER_TILE = 40
E_PER_TILE = ROWS_PER_TILE * 128
E_PAD = NW * E_PER_TILE
ROWS_TOTAL = E_PAD // 128
BIG_SCORE = 40.0
K_FIRE = 8


def _mesh():
    return plsc.VectorSubcoreMesh(
        core_axis_name="c", subcore_axis_name="s",
        num_cores=NUM_CORES, num_subcores=NUM_SUBCORES)


def _wid():
    return lax.axis_index("s") * NUM_CORES + lax.axis_index("c")


def _compute_keys(src_v, dst_v, mask_v, keys_v, vals_v, valid_v, base_eid,
                  dummy):
    def row(g, _):
        def chunk(cc, _):
            off = pl.multiple_of(cc * 16, 16)
            s = src_v[g, pl.ds(off, 16)]
            d = dst_v[g, pl.ds(off, 16)]
            ms = plsc.load_gather(mask_v, [s])
            md = plsc.load_gather(mask_v, [d])
            vm = ms != md
            keys_v[g, pl.ds(off, 16)] = jnp.where(vm, s * N_NODES + d, dummy)
            if vals_v is not None:
                vals_v[g, pl.ds(off, 16)] = (
                    base_eid + g * 128 + cc * 16 + lax.iota(jnp.int32, 16))
            if valid_v is not None:
                valid_v[g, pl.ds(off, 16)] = vm.astype(jnp.int32)
                src_v[g, pl.ds(off, 16)] = jnp.where(vm, s, 0)
                dst_v[g, pl.ds(off, 16)] = jnp.where(vm, d, 0)
            return _
        return lax.fori_loop(0, 8, chunk, None)
    lax.fori_loop(0, ROWS_PER_TILE, row, None)


@functools.partial(
    pl.kernel,
    out_type=jax.ShapeDtypeStruct((TABLE_PAD,), jnp.int32),
    mesh=_mesh(),
    compiler_params=pltpu.CompilerParams(needs_layout_passes=False),
    scratch_types=[
        pltpu.VMEM((ROWS_PER_TILE, 128), jnp.int32),
        pltpu.VMEM((ROWS_PER_TILE, 128), jnp.int32),
        pltpu.VMEM((ROWS_PER_TILE, 128), jnp.int32),
        pltpu.VMEM((ROWS_PER_TILE, 128), jnp.int32),
        pltpu.VMEM((N_NODES,), jnp.int32),
        pltpu.SemaphoreType.DMA,
    ],
)
def _scatter_ids(src_hbm, dst_hbm, mask_hbm, table_hbm, src_v, dst_v, keys_v,
                 vals_v, mask_v, sem):
    wid = _wid()
    row0 = wid * ROWS_PER_TILE
    pltpu.sync_copy(src_hbm.at[pl.ds(row0, ROWS_PER_TILE)], src_v)
    pltpu.sync_copy(dst_hbm.at[pl.ds(row0, ROWS_PER_TILE)], dst_v)
    pltpu.sync_copy(mask_hbm, mask_v)
    _compute_keys(src_v, dst_v, mask_v, keys_v, vals_v, None, row0 * 128,
                  TABLE_SIZE + wid * 16)

    cps = [pltpu.async_copy(vals_v.at[g], table_hbm.at[keys_v.at[g]], sem)
           for g in range(ROWS_PER_TILE)]
    for c in cps:
        c.wait()


@functools.partial(
    pl.kernel,
    out_type=jax.ShapeDtypeStruct((ROWS_TOTAL, 128), jnp.float32),
    mesh=_mesh(),
    compiler_params=pltpu.CompilerParams(needs_layout_passes=False),
    scratch_types=[
        pltpu.VMEM((ROWS_PER_TILE, 128), jnp.int32),
        pltpu.VMEM((ROWS_PER_TILE, 128), jnp.int32),
        pltpu.VMEM((ROWS_PER_TILE, 128), jnp.int32),
        pltpu.VMEM((ROWS_PER_TILE, 128), jnp.int32),
        pltpu.VMEM((ROWS_PER_TILE, 128), jnp.int32),
        pltpu.VMEM((ROWS_PER_TILE, 128), jnp.float32),
        pltpu.VMEM((N_NODES,), jnp.int32),
        pltpu.VMEM((128, D_FEAT), jnp.float32),
        pltpu.VMEM((128, D_FEAT), jnp.float32),
        pltpu.VMEM((128, D_FEAT), jnp.float32),
        pltpu.VMEM((128, D_FEAT), jnp.float32),
        pltpu.SemaphoreType.DMA,
        pltpu.SemaphoreType.DMA,
        pltpu.SemaphoreType.DMA,
        pltpu.SemaphoreType.DMA,
        pltpu.SemaphoreType.DMA,
    ],
)
def _gather_dot(src_hbm, dst_hbm, mask_hbm, a_hbm, table_hbm, out_hbm,
                src_v, dst_v, keys_v, rep_v, valid_v, scores_v, mask_v,
                rows_s0, rows_d0, rows_s1, rows_d1,
                sem_a, sem_a0, sem_b0, sem_a1, sem_b1):
    wid = _wid()
    row0 = wid * ROWS_PER_TILE
    base_eid = row0 * 128
    pltpu.sync_copy(src_hbm.at[pl.ds(row0, ROWS_PER_TILE)], src_v)
    pltpu.sync_copy(dst_hbm.at[pl.ds(row0, ROWS_PER_TILE)], dst_v)
    pltpu.sync_copy(mask_hbm, mask_v)
    _compute_keys(src_v, dst_v, mask_v, keys_v, None, valid_v, base_eid,
                  TABLE_SIZE + wid * 16)

    rep_cps = [
        pltpu.async_copy(table_hbm.at[keys_v.at[g]], rep_v.at[g], sem_a)
        for g in range(ROWS_PER_TILE)
    ]

    def start_grp(g, bs, bd, sa, sb):
        pltpu.async_copy(a_hbm.at[src_v.at[g]], bs, sa)
        pltpu.async_copy(a_hbm.at[dst_v.at[g]], bd, sb)

    def wait_grp(g, bs, bd, sa, sb):
        pltpu.make_async_copy(a_hbm.at[src_v.at[g]], bs, sa).wait()
        pltpu.make_async_copy(a_hbm.at[dst_v.at[g]], bd, sb).wait()

    def compute_grp(g, rows_s, rows_d):
        def sub(bb, _):
            b0 = pl.multiple_of(bb * 16, 16)
            eidx = b0 + lax.iota(jnp.int32, 16)
            accs = [jnp.zeros((16,), jnp.float32) for _ in range(4)]
            for d in range(0, D_FEAT, 4):
                for k in range(4):
                    dsplat = jnp.full((16,), d + k, jnp.int32)
                    gs = plsc.load_gather(rows_s, [eidx, dsplat])
                    gd = plsc.load_gather(rows_d, [eidx, dsplat])
                    accs[k] = accs[k] + gs * gd
            sv = (accs[0] + accs[1]) + (accs[2] + accs[3])
            eid = (base_eid + g * 128 + bb * 16 + lax.iota(jnp.int32, 16))
            rep = rep_v[g, pl.ds(b0, 16)]
            valid = (rep == eid) & (valid_v[g, pl.ds(b0, 16)] != 0)
            scores_v[g, pl.ds(b0, 16)] = jnp.where(valid, sv, BIG_SCORE)
            return _
        lax.fori_loop(0, 8, sub, None)

    start_grp(0, rows_s0, rows_d0, sem_a0, sem_b0)
    for c in rep_cps:
        c.wait()

    def grp2(gg, _):
        g0 = gg * 2
        g1 = g0 + 1
        start_grp(g1, rows_s1, rows_d1, sem_a1, sem_b1)
        wait_grp(g0, rows_s0, rows_d0, sem_a0, sem_b0)
        compute_grp(g0, rows_s0, rows_d0)

        @pl.when(g1 + 1 < ROWS_PER_TILE)
        def _prefetch():
            start_grp(g1 + 1, rows_s0, rows_d0, sem_a0, sem_b0)

        wait_grp(g1, rows_s1, rows_d1, sem_a1, sem_b1)
        compute_grp(g1, rows_s1, rows_d1)
        return _
    lax.fori_loop(0, ROWS_PER_TILE // 2, grp2, None)
    pltpu.sync_copy(scores_v, out_hbm.at[pl.ds(row0, ROWS_PER_TILE)])


def _tc_loss_body(scores_ref, out_ref):
    s = scores_ref[...]
    terms = -jnp.log(jax.nn.sigmoid(s) + EPS)
    out_ref[0, 0] = jnp.sum(terms)


_tc_loss = pl.pallas_call(
    _tc_loss_body,
    out_shape=jax.ShapeDtypeStruct((1, 1), jnp.float32),
    out_specs=pl.BlockSpec(memory_space=pltpu.SMEM),
)


def kernel(A_star, edge_index, node_mask):
    ei = edge_index.astype(jnp.int32)
    src = jnp.pad(ei[0], (0, E_PAD - N_EDGES)).reshape(ROWS_TOTAL, 128)
    dst = jnp.pad(ei[1], (0, E_PAD - N_EDGES)).reshape(ROWS_TOTAL, 128)
    mask_i = node_mask.astype(jnp.int32)
    table = _scatter_ids(src, dst, mask_i)
    scores = _gather_dot(src, dst, mask_i, A_star, table)
    return _tc_loss(scores)[0, 0]


# speedup vs baseline: 9.6698x; 9.6698x over previous
import functools

import jax
import jax.numpy as jnp
from jax import lax
from jax.experimental import pallas as pl
from jax.experimental.pallas import tpu as pltpu
from jax.experimental.pallas import tpu_sc as plsc

N_NODES = 10000
D_FEAT = 128
N_EDGES = 160000
EPS = 1e-15
TABLE_SIZE = N_NODES * N_NODES
TABLE_PAD = TABLE_SIZE + 163840

NUM_CORES = 2
NUM_SUBCORES = 16
NW = NUM_CORES * NUM_SUBCORES
ROWS_PER_TILE = 40
E_PER_TILE = ROWS_PER_TILE * 128
E_PAD = NW * E_PER_TILE
ROWS_TOTAL = E_PAD // 128
BIG_SCORE = 40.0
K_FIRE = 8


def _mesh():
    return plsc.VectorSubcoreMesh(
        core_axis_name="c", subcore_axis_name="s",
        num_cores=NUM_CORES, num_subcores=NUM_SUBCORES)


def _wid():
    return lax.axis_index("s") * NUM_CORES + lax.axis_index("c")


def _compute_keys(src_v, dst_v, mask_v, keys_v, vals_v, valid_v, base_eid):
    def row(g, _):
        def chunk(cc, _):
            off = pl.multiple_of(cc * 16, 16)
            s = src_v[g, pl.ds(off, 16)]
            d = dst_v[g, pl.ds(off, 16)]
            ms = plsc.load_gather(mask_v, [s])
            md = plsc.load_gather(mask_v, [d])
            vm = ms != md
            eid = base_eid + g * 128 + cc * 16 + lax.iota(jnp.int32, 16)
            keys_v[g, pl.ds(off, 16)] = jnp.where(
                vm, s * N_NODES + d, TABLE_SIZE + eid)
            if vals_v is not None:
                vals_v[g, pl.ds(off, 16)] = eid
            if valid_v is not None:
                valid_v[g, pl.ds(off, 16)] = vm.astype(jnp.int32)
            return _
        return lax.fori_loop(0, 8, chunk, None)
    lax.fori_loop(0, ROWS_PER_TILE, row, None)


@functools.partial(
    pl.kernel,
    out_type=jax.ShapeDtypeStruct((TABLE_PAD,), jnp.int32),
    mesh=_mesh(),
    compiler_params=pltpu.CompilerParams(needs_layout_passes=False),
    scratch_types=[
        pltpu.VMEM((ROWS_PER_TILE, 128), jnp.int32),
        pltpu.VMEM((ROWS_PER_TILE, 128), jnp.int32),
        pltpu.VMEM((ROWS_PER_TILE, 128), jnp.int32),
        pltpu.VMEM((ROWS_PER_TILE, 128), jnp.int32),
        pltpu.VMEM((N_NODES,), jnp.int32),
        pltpu.SemaphoreType.DMA,
    ],
)
def _scatter_ids(src_hbm, dst_hbm, mask_hbm, table_hbm, src_v, dst_v, keys_v,
                 vals_v, mask_v, sem):
    wid = _wid()
    row0 = wid * ROWS_PER_TILE
    pltpu.sync_copy(src_hbm.at[pl.ds(row0, ROWS_PER_TILE)], src_v)
    pltpu.sync_copy(dst_hbm.at[pl.ds(row0, ROWS_PER_TILE)], dst_v)
    pltpu.sync_copy(mask_hbm, mask_v)
    _compute_keys(src_v, dst_v, mask_v, keys_v, vals_v, None, row0 * 128)

    cps = [pltpu.async_copy(vals_v.at[g], table_hbm.at[keys_v.at[g]], sem)
           for g in range(ROWS_PER_TILE)]
    for c in cps:
        c.wait()


@functools.partial(
    pl.kernel,
    out_type=jax.ShapeDtypeStruct((ROWS_TOTAL, 128), jnp.float32),
    mesh=_mesh(),
    compiler_params=pltpu.CompilerParams(needs_layout_passes=False),
    scratch_types=[
        pltpu.VMEM((ROWS_PER_TILE, 128), jnp.int32),
        pltpu.VMEM((ROWS_PER_TILE, 128), jnp.int32),
        pltpu.VMEM((ROWS_PER_TILE, 128), jnp.int32),
        pltpu.VMEM((ROWS_PER_TILE, 128), jnp.int32),
        pltpu.VMEM((ROWS_PER_TILE, 128), jnp.int32),
        pltpu.VMEM((ROWS_PER_TILE, 128), jnp.float32),
        pltpu.VMEM((N_NODES,), jnp.int32),
        pltpu.VMEM((128, D_FEAT), jnp.float32),
        pltpu.VMEM((128, D_FEAT), jnp.float32),
        pltpu.VMEM((128, D_FEAT), jnp.float32),
        pltpu.VMEM((128, D_FEAT), jnp.float32),
        pltpu.SemaphoreType.DMA,
        pltpu.SemaphoreType.DMA,
        pltpu.SemaphoreType.DMA,
        pltpu.SemaphoreType.DMA,
        pltpu.SemaphoreType.DMA,
    ],
)
def _gather_dot(src_hbm, dst_hbm, mask_hbm, a_hbm, table_hbm, out_hbm,
                src_v, dst_v, keys_v, rep_v, valid_v, scores_v, mask_v,
                rows_s0, rows_d0, rows_s1, rows_d1,
                sem_a, sem_a0, sem_b0, sem_a1, sem_b1):
    wid = _wid()
    row0 = wid * ROWS_PER_TILE
    base_eid = row0 * 128
    pltpu.sync_copy(src_hbm.at[pl.ds(row0, ROWS_PER_TILE)], src_v)
    pltpu.sync_copy(dst_hbm.at[pl.ds(row0, ROWS_PER_TILE)], dst_v)
    pltpu.sync_copy(mask_hbm, mask_v)
    _compute_keys(src_v, dst_v, mask_v, keys_v, None, valid_v, base_eid)

    rep_cps = [
        pltpu.async_copy(table_hbm.at[keys_v.at[g]], rep_v.at[g], sem_a)
        for g in range(ROWS_PER_TILE)
    ]

    def start_grp(g, bs, bd, sa, sb):
        pltpu.async_copy(a_hbm.at[src_v.at[g]], bs, sa)
        pltpu.async_copy(a_hbm.at[dst_v.at[g]], bd, sb)

    def wait_grp(g, bs, bd, sa, sb):
        pltpu.make_async_copy(a_hbm.at[src_v.at[g]], bs, sa).wait()
        pltpu.make_async_copy(a_hbm.at[dst_v.at[g]], bd, sb).wait()

    def compute_grp(g, rows_s, rows_d):
        def sub(bb, _):
            b0 = pl.multiple_of(bb * 16, 16)
            eidx = b0 + lax.iota(jnp.int32, 16)
            accs = [jnp.zeros((16,), jnp.float32) for _ in range(4)]
            for d in range(0, D_FEAT, 4):
                for k in range(4):
                    dsplat = jnp.full((16,), d + k, jnp.int32)
                    gs = plsc.load_gather(rows_s, [eidx, dsplat])
                    gd = plsc.load_gather(rows_d, [eidx, dsplat])
                    accs[k] = accs[k] + gs * gd
            sv = (accs[0] + accs[1]) + (accs[2] + accs[3])
            eid = (base_eid + g * 128 + bb * 16 + lax.iota(jnp.int32, 16))
            rep = rep_v[g, pl.ds(b0, 16)]
            valid = (rep == eid) & (valid_v[g, pl.ds(b0, 16)] != 0)
            scores_v[g, pl.ds(b0, 16)] = jnp.where(valid, sv, BIG_SCORE)
            return _
        lax.fori_loop(0, 8, sub, None)

    start_grp(0, rows_s0, rows_d0, sem_a0, sem_b0)
    for c in rep_cps:
        c.wait()

    def grp2(gg, _):
        g0 = gg * 2
        g1 = g0 + 1
        start_grp(g1, rows_s1, rows_d1, sem_a1, sem_b1)
        wait_grp(g0, rows_s0, rows_d0, sem_a0, sem_b0)
        compute_grp(g0, rows_s0, rows_d0)

        @pl.when(g1 + 1 < ROWS_PER_TILE)
        def _prefetch():
            start_grp(g1 + 1, rows_s0, rows_d0, sem_a0, sem_b0)

        wait_grp(g1, rows_s1, rows_d1, sem_a1, sem_b1)
        compute_grp(g1, rows_s1, rows_d1)
        return _
    lax.fori_loop(0, ROWS_PER_TILE // 2, grp2, None)
    pltpu.sync_copy(scores_v, out_hbm.at[pl.ds(row0, ROWS_PER_TILE)])


def _tc_loss_body(scores_ref, out_ref):
    s = scores_ref[...]
    terms = -jnp.log(jax.nn.sigmoid(s) + EPS)
    out_ref[0, 0] = jnp.sum(terms)


_tc_loss = pl.pallas_call(
    _tc_loss_body,
    out_shape=jax.ShapeDtypeStruct((1, 1), jnp.float32),
    out_specs=pl.BlockSpec(memory_space=pltpu.SMEM),
)


def kernel(A_star, edge_index, node_mask):
    ei = edge_index.astype(jnp.int32)
    src = jnp.pad(ei[0], (0, E_PAD - N_EDGES)).reshape(ROWS_TOTAL, 128)
    dst = jnp.pad(ei[1], (0, E_PAD - N_EDGES)).reshape(ROWS_TOTAL, 128)
    mask_i = node_mask.astype(jnp.int32)
    table = _scatter_ids(src, dst, mask_i)
    scores = _gather_dot(src, dst, mask_i, A_star, table)
    return _tc_loss(scores)[0, 0]


# speedup vs baseline: 12.3805x; 1.2803x over previous
import functools

import jax
import jax.numpy as jnp
from jax import lax
from jax.experimental import pallas as pl
from jax.experimental.pallas import tpu as pltpu
from jax.experimental.pallas import tpu_sc as plsc

N_NODES = 10000
D_FEAT = 128
N_EDGES = 160000
EPS = 1e-15
TABLE_SIZE = N_NODES * N_NODES
TABLE_PAD = TABLE_SIZE + 163840

NUM_CORES = 2
NUM_SUBCORES = 16
NW = NUM_CORES * NUM_SUBCORES
ROWS_PER_TILE = 40
E_PER_TILE = ROWS_PER_TILE * 128
E_PAD = NW * E_PER_TILE
ROWS_TOTAL = E_PAD // 128
BIG_SCORE = 40.0
K_FIRE = 8


def _mesh():
    return plsc.VectorSubcoreMesh(
        core_axis_name="c", subcore_axis_name="s",
        num_cores=NUM_CORES, num_subcores=NUM_SUBCORES)


def _wid():
    return lax.axis_index("s") * NUM_CORES + lax.axis_index("c")


def _compute_keys(src_v, dst_v, mask_v, keys_v, vals_v, valid_v, base_eid):
    def row(g, _):
        def chunk(cc, _):
            off = pl.multiple_of(cc * 16, 16)
            s = src_v[g, pl.ds(off, 16)]
            d = dst_v[g, pl.ds(off, 16)]
            ms = plsc.load_gather(mask_v, [s])
            md = plsc.load_gather(mask_v, [d])
            vm = ms != md
            eid = base_eid + g * 128 + cc * 16 + lax.iota(jnp.int32, 16)
            keys_v[g, pl.ds(off, 16)] = jnp.where(
                vm, s * N_NODES + d, TABLE_SIZE + eid)
            if vals_v is not None:
                vals_v[g, pl.ds(off, 16)] = eid
            if valid_v is not None:
                valid_v[g, pl.ds(off, 16)] = vm.astype(jnp.int32)
            return _
        return lax.fori_loop(0, 8, chunk, None)
    lax.fori_loop(0, ROWS_PER_TILE, row, None)


@functools.partial(
    pl.kernel,
    out_type=jax.ShapeDtypeStruct((TABLE_PAD,), jnp.int32),
    mesh=_mesh(),
    compiler_params=pltpu.CompilerParams(needs_layout_passes=False),
    scratch_types=[
        pltpu.VMEM((ROWS_PER_TILE, 128), jnp.int32),
        pltpu.VMEM((ROWS_PER_TILE, 128), jnp.int32),
        pltpu.VMEM((ROWS_PER_TILE, 128), jnp.int32),
        pltpu.VMEM((ROWS_PER_TILE, 128), jnp.int32),
        pltpu.VMEM((N_NODES,), jnp.int32),
        pltpu.SemaphoreType.DMA,
    ],
)
def _scatter_ids(src_hbm, dst_hbm, mask_hbm, table_hbm, src_v, dst_v, keys_v,
                 vals_v, mask_v, sem):
    wid = _wid()
    row0 = wid * ROWS_PER_TILE
    pltpu.sync_copy(src_hbm.at[pl.ds(row0, ROWS_PER_TILE)], src_v)
    pltpu.sync_copy(dst_hbm.at[pl.ds(row0, ROWS_PER_TILE)], dst_v)
    pltpu.sync_copy(mask_hbm, mask_v)
    _compute_keys(src_v, dst_v, mask_v, keys_v, vals_v, None, row0 * 128)

    cps = [pltpu.async_copy(vals_v.at[g], table_hbm.at[keys_v.at[g]], sem)
           for g in range(ROWS_PER_TILE)]
    for c in cps:
        c.wait()


@functools.partial(
    pl.kernel,
    out_type=jax.ShapeDtypeStruct((ROWS_TOTAL, 128), jnp.float32),
    mesh=_mesh(),
    compiler_params=pltpu.CompilerParams(needs_layout_passes=False),
    scratch_types=[
        pltpu.VMEM((ROWS_PER_TILE, 128), jnp.int32),
        pltpu.VMEM((ROWS_PER_TILE, 128), jnp.int32),
        pltpu.VMEM((ROWS_PER_TILE, 128), jnp.int32),
        pltpu.VMEM((ROWS_PER_TILE, 128), jnp.int32),
        pltpu.VMEM((ROWS_PER_TILE, 128), jnp.int32),
        pltpu.VMEM((ROWS_PER_TILE, 128), jnp.float32),
        pltpu.VMEM((N_NODES,), jnp.int32),
        pltpu.VMEM((128, D_FEAT), jnp.float32),
        pltpu.VMEM((128, D_FEAT), jnp.float32),
        pltpu.VMEM((128, D_FEAT), jnp.float32),
        pltpu.VMEM((128, D_FEAT), jnp.float32),
        pltpu.SemaphoreType.DMA,
        pltpu.SemaphoreType.DMA,
        pltpu.SemaphoreType.DMA,
        pltpu.SemaphoreType.DMA,
        pltpu.SemaphoreType.DMA,
    ],
)
def _gather_dot(src_hbm, dst_hbm, mask_hbm, a_hbm, table_hbm, out_hbm,
                src_v, dst_v, keys_v, rep_v, valid_v, scores_v, mask_v,
                rows_s0, rows_d0, rows_s1, rows_d1,
                sem_a, sem_a0, sem_b0, sem_a1, sem_b1):
    wid = _wid()
    row0 = wid * ROWS_PER_TILE
    base_eid = row0 * 128
    pltpu.sync_copy(src_hbm.at[pl.ds(row0, ROWS_PER_TILE)], src_v)
    pltpu.sync_copy(dst_hbm.at[pl.ds(row0, ROWS_PER_TILE)], dst_v)
    pltpu.sync_copy(mask_hbm, mask_v)
    _compute_keys(src_v, dst_v, mask_v, keys_v, None, valid_v, base_eid)

    rep_cps = [
        pltpu.async_copy(table_hbm.at[keys_v.at[g]], rep_v.at[g], sem_a)
        for g in range(ROWS_PER_TILE)
    ]

    def start_grp(g, bs, bd, sa, sb):
        pltpu.async_copy(a_hbm.at[src_v.at[g]], bs, sa)
        pltpu.async_copy(a_hbm.at[dst_v.at[g]], bd, sb)

    def wait_grp(g, bs, bd, sa, sb):
        pltpu.make_async_copy(a_hbm.at[src_v.at[g]], bs, sa).wait()
        pltpu.make_async_copy(a_hbm.at[dst_v.at[g]], bd, sb).wait()

    def compute_grp(g, rows_s, rows_d):
        def sub(bb, _):
            b0 = pl.multiple_of(bb * 16, 16)
            iot = lax.iota(jnp.int32, 16)
            eidx = b0 + iot
            accs = [jnp.zeros((16,), jnp.float32) for _ in range(4)]
            for c in range(8):
                for j in range(16):
                    dvec = ((iot + j) & 15) + c * 16
                    gs = plsc.load_gather(rows_s, [eidx, dvec])
                    gd = plsc.load_gather(rows_d, [eidx, dvec])
                    accs[j & 3] = accs[j & 3] + gs * gd
            sv = (accs[0] + accs[1]) + (accs[2] + accs[3])
            eid = (base_eid + g * 128 + bb * 16 + lax.iota(jnp.int32, 16))
            rep = rep_v[g, pl.ds(b0, 16)]
            valid = (rep == eid) & (valid_v[g, pl.ds(b0, 16)] != 0)
            scores_v[g, pl.ds(b0, 16)] = jnp.where(valid, sv, BIG_SCORE)
            return _
        lax.fori_loop(0, 8, sub, None)

    start_grp(0, rows_s0, rows_d0, sem_a0, sem_b0)
    for c in rep_cps:
        c.wait()

    def grp2(gg, _):
        g0 = gg * 2
        g1 = g0 + 1
        start_grp(g1, rows_s1, rows_d1, sem_a1, sem_b1)
        wait_grp(g0, rows_s0, rows_d0, sem_a0, sem_b0)
        compute_grp(g0, rows_s0, rows_d0)

        @pl.when(g1 + 1 < ROWS_PER_TILE)
        def _prefetch():
            start_grp(g1 + 1, rows_s0, rows_d0, sem_a0, sem_b0)

        wait_grp(g1, rows_s1, rows_d1, sem_a1, sem_b1)
        compute_grp(g1, rows_s1, rows_d1)
        return _
    lax.fori_loop(0, ROWS_PER_TILE // 2, grp2, None)
    pltpu.sync_copy(scores_v, out_hbm.at[pl.ds(row0, ROWS_PER_TILE)])


def _tc_loss_body(scores_ref, out_ref):
    s = scores_ref[...]
    terms = -jnp.log(jax.nn.sigmoid(s) + EPS)
    out_ref[0, 0] = jnp.sum(terms)


_tc_loss = pl.pallas_call(
    _tc_loss_body,
    out_shape=jax.ShapeDtypeStruct((1, 1), jnp.float32),
    out_specs=pl.BlockSpec(memory_space=pltpu.SMEM),
)


def kernel(A_star, edge_index, node_mask):
    ei = edge_index.astype(jnp.int32)
    src = jnp.pad(ei[0], (0, E_PAD - N_EDGES)).reshape(ROWS_TOTAL, 128)
    dst = jnp.pad(ei[1], (0, E_PAD - N_EDGES)).reshape(ROWS_TOTAL, 128)
    mask_i = node_mask.astype(jnp.int32)
    table = _scatter_ids(src, dst, mask_i)
    scores = _gather_dot(src, dst, mask_i, A_star, table)
    return _tc_loss(scores)[0, 0]


# speedup vs baseline: 35.3951x; 2.8589x over previous
import functools

import jax
import jax.numpy as jnp
from jax import lax
from jax.experimental import pallas as pl
from jax.experimental.pallas import tpu as pltpu
from jax.experimental.pallas import tpu_sc as plsc

N_NODES = 10000
D_FEAT = 128
N_EDGES = 160000
EPS = 1e-15
TABLE_SIZE = N_NODES * N_NODES
TABLE_PAD = TABLE_SIZE + 2 * 163840

NUM_CORES = 2
NUM_SUBCORES = 16
NW = NUM_CORES * NUM_SUBCORES
ROWS_PER_TILE = 40
E_PER_TILE = ROWS_PER_TILE * 128
E_PAD = NW * E_PER_TILE
ROWS_TOTAL = E_PAD // 128
BIG_SCORE = 40.0


def _mesh():
    return plsc.VectorSubcoreMesh(
        core_axis_name="c", subcore_axis_name="s",
        num_cores=NUM_CORES, num_subcores=NUM_SUBCORES)


def _wid():
    return lax.axis_index("s") * NUM_CORES + lax.axis_index("c")


def _compact(sbuf, dbuf, kbuf, mask_v):
    iot = lax.iota(jnp.int32, 16)

    def grp(g, n):
        def chunk(cc, n):
            off = pl.multiple_of(cc * 16, 16)
            s = sbuf[pl.ds(g * 128 + off, 16)]
            d = dbuf[pl.ds(g * 128 + off, 16)]
            ms = plsc.load_gather(mask_v, [s])
            md = plsc.load_gather(mask_v, [d])
            vm = ms != md
            plsc.store_compressed(sbuf.at[pl.ds(n, 16)], s, mask=vm)
            plsc.store_compressed(dbuf.at[pl.ds(n, 16)], d, mask=vm)
            plsc.store_compressed(kbuf.at[pl.ds(n, 16)], s * N_NODES + d,
                                  mask=vm)
            return n + plsc.all_reduce_population_count(vm)[0]
        return lax.fori_loop(0, 8, chunk, n)
    return lax.fori_loop(0, ROWS_PER_TILE, grp, jnp.int32(0))


def _fix_tail(sbuf, dbuf, kbuf, nvalid, ngrp, base_eid):
    iot = lax.iota(jnp.int32, 16)

    @pl.when(ngrp > 0)
    def _():
        t0 = (ngrp - 1) * 128
        def chunk(cc, _):
            off = pl.multiple_of(cc * 16, 16)
            pos = t0 + cc * 16 + iot
            pad = pos >= nvalid
            spread = t0 + cc * 16 + iot
            s_old = sbuf[pl.ds(t0 + off, 16)]
            d_old = dbuf[pl.ds(t0 + off, 16)]
            k_old = kbuf[pl.ds(t0 + off, 16)]
            sbuf[pl.ds(t0 + off, 16)] = jnp.where(pad, spread, s_old)
            dbuf[pl.ds(t0 + off, 16)] = jnp.where(pad, spread, d_old)
            kbuf[pl.ds(t0 + off, 16)] = jnp.where(
                pad, TABLE_SIZE + 163840 + base_eid + spread, k_old)
            return _
        lax.fori_loop(0, 8, chunk, None)


@functools.partial(
    pl.kernel,
    out_type=jax.ShapeDtypeStruct((TABLE_PAD,), jnp.int32),
    mesh=_mesh(),
    compiler_params=pltpu.CompilerParams(needs_layout_passes=False),
    scratch_types=[
        pltpu.VMEM((E_PER_TILE + 16,), jnp.int32),
        pltpu.VMEM((E_PER_TILE + 16,), jnp.int32),
        pltpu.VMEM((E_PER_TILE + 16,), jnp.int32),
        pltpu.VMEM((ROWS_PER_TILE, 128), jnp.int32),
        pltpu.VMEM((ROWS_PER_TILE, 128), jnp.int32),
        pltpu.VMEM((N_NODES,), jnp.int32),
        pltpu.SemaphoreType.DMA,
    ],
)
def _scatter_ids(src_hbm, dst_hbm, mask_hbm, table_hbm,
                 sbuf, dbuf, kbuf, keys2d, vals_v, mask_v, sem):
    wid = _wid()
    base_eid = wid * E_PER_TILE
    pltpu.sync_copy(src_hbm.at[pl.ds(base_eid, E_PER_TILE)],
                    sbuf.at[pl.ds(0, E_PER_TILE)])
    pltpu.sync_copy(dst_hbm.at[pl.ds(base_eid, E_PER_TILE)],
                    dbuf.at[pl.ds(0, E_PER_TILE)])
    pltpu.sync_copy(mask_hbm, mask_v)
    iot = lax.iota(jnp.int32, 16)

    nvalid = _compact(sbuf, dbuf, kbuf, mask_v)
    ngrp = (nvalid + 127) // 128
    _fix_tail(sbuf, dbuf, kbuf, nvalid, ngrp, base_eid)

    def to2d(g, _):
        def chunk(cc, _):
            off = pl.multiple_of(cc * 16, 16)
            keys2d[g, pl.ds(off, 16)] = kbuf[pl.ds(g * 128 + off, 16)]
            vals_v[g, pl.ds(off, 16)] = base_eid + g * 128 + cc * 16 + iot
            return _
        return lax.fori_loop(0, 8, chunk, None)
    lax.fori_loop(0, ROWS_PER_TILE, to2d, None)

    def scat_start(g, _):
        pltpu.async_copy(vals_v.at[g], table_hbm.at[keys2d.at[g]], sem)
        return _
    lax.fori_loop(0, ngrp, scat_start, None)

    def scat_drain(g, _):
        pltpu.make_async_copy(vals_v.at[g], table_hbm.at[keys2d.at[g]],
                              sem).wait()
        return _
    lax.fori_loop(0, ngrp, scat_drain, None)


@functools.partial(
    pl.kernel,
    out_type=jax.ShapeDtypeStruct((E_PAD,), jnp.float32),
    mesh=_mesh(),
    compiler_params=pltpu.CompilerParams(needs_layout_passes=False),
    scratch_types=[
        pltpu.VMEM((E_PER_TILE + 16,), jnp.int32),
        pltpu.VMEM((E_PER_TILE + 16,), jnp.int32),
        pltpu.VMEM((E_PER_TILE + 16,), jnp.int32),
        pltpu.VMEM((ROWS_PER_TILE, 128), jnp.int32),
        pltpu.VMEM((E_PER_TILE,), jnp.float32),
        pltpu.VMEM((N_NODES,), jnp.int32),
        pltpu.VMEM((128, D_FEAT), jnp.float32),
        pltpu.VMEM((128, D_FEAT), jnp.float32),
        pltpu.VMEM((128, D_FEAT), jnp.float32),
        pltpu.VMEM((128, D_FEAT), jnp.float32),
        pltpu.SemaphoreType.DMA,
        pltpu.SemaphoreType.DMA,
        pltpu.SemaphoreType.DMA,
        pltpu.SemaphoreType.DMA,
        pltpu.SemaphoreType.DMA,
    ],
)
def _gather_dot(src_hbm, dst_hbm, mask_hbm, a_hbm, table_hbm, out_hbm,
                sbuf, dbuf, kbuf, rep_v, scores_v, mask_v,
                rows_s0, rows_d0, rows_s1, rows_d1,
                sem_a, sem_a0, sem_b0, sem_a1, sem_b1):
    wid = _wid()
    base_eid = wid * E_PER_TILE
    pltpu.sync_copy(src_hbm.at[pl.ds(base_eid, E_PER_TILE)],
                    sbuf.at[pl.ds(0, E_PER_TILE)])
    pltpu.sync_copy(dst_hbm.at[pl.ds(base_eid, E_PER_TILE)],
                    dbuf.at[pl.ds(0, E_PER_TILE)])
    pltpu.sync_copy(mask_hbm, mask_v)
    iot = lax.iota(jnp.int32, 16)

    nvalid = _compact(sbuf, dbuf, kbuf, mask_v)
    ngrp = (nvalid + 127) // 128
    _fix_tail(sbuf, dbuf, kbuf, nvalid, ngrp, base_eid)

    def prefill(g, _):
        def chunk(cc, _):
            off = pl.multiple_of(cc * 16, 16)
            scores_v[pl.ds(g * 128 + off, 16)] = jnp.full(
                (16,), BIG_SCORE, jnp.float32)
            return _
        return lax.fori_loop(0, 8, chunk, None)
    lax.fori_loop(0, ROWS_PER_TILE, prefill, None)

    def rep_start(g, _):
        pltpu.async_copy(table_hbm.at[kbuf.at[pl.ds(g * 128, 128)]],
                         rep_v.at[g], sem_a)
        return _
    lax.fori_loop(0, ngrp, rep_start, None)

    def start_grp(g, bs, bd, sa, sb):
        pltpu.async_copy(a_hbm.at[sbuf.at[pl.ds(g * 128, 128)]], bs, sa)
        pltpu.async_copy(a_hbm.at[dbuf.at[pl.ds(g * 128, 128)]], bd, sb)

    def wait_grp(g, bs, bd, sa, sb):
        pltpu.make_async_copy(
            a_hbm.at[sbuf.at[pl.ds(g * 128, 128)]], bs, sa).wait()
        pltpu.make_async_copy(
            a_hbm.at[dbuf.at[pl.ds(g * 128, 128)]], bd, sb).wait()

    def compute_grp(g, rows_s, rows_d):
        def sub(bb, _):
            b0 = pl.multiple_of(bb * 16, 16)
            eidx = b0 + iot
            accs = [jnp.zeros((16,), jnp.float32) for _ in range(4)]
            for c in range(8):
                for j in range(16):
                    dvec = ((iot + j) & 15) + c * 16
                    gs = plsc.load_gather(rows_s, [eidx, dvec])
                    gd = plsc.load_gather(rows_d, [eidx, dvec])
                    accs[j & 3] = accs[j & 3] + gs * gd
            sv = (accs[0] + accs[1]) + (accs[2] + accs[3])
            cid = base_eid + g * 128 + b0 + iot
            rep = rep_v[g, pl.ds(b0, 16)]
            valid = (rep == cid) & ((g * 128 + b0 + iot) < nvalid)
            scores_v[pl.ds(g * 128 + b0, 16)] = jnp.where(
                valid, sv, BIG_SCORE)
            return _
        lax.fori_loop(0, 8, sub, None)

    @pl.when(ngrp > 0)
    def _prologue():
        start_grp(0, rows_s0, rows_d0, sem_a0, sem_b0)

    def rep_drain(g, _):
        pltpu.make_async_copy(
            table_hbm.at[kbuf.at[pl.ds(g * 128, 128)]], rep_v.at[g],
            sem_a).wait()
        return _
    lax.fori_loop(0, ngrp, rep_drain, None)

    def grp2(gg, _):
        g0 = gg * 2
        g1 = g0 + 1
        start_grp(g1, rows_s1, rows_d1, sem_a1, sem_b1)
        wait_grp(g0, rows_s0, rows_d0, sem_a0, sem_b0)
        compute_grp(g0, rows_s0, rows_d0)

        @pl.when(g0 + 2 < ngrp)
        def _prefetch():
            start_grp(g0 + 2, rows_s0, rows_d0, sem_a0, sem_b0)

        wait_grp(g1, rows_s1, rows_d1, sem_a1, sem_b1)
        compute_grp(g1, rows_s1, rows_d1)
        return _
    lax.fori_loop(0, ngrp // 2, grp2, None)

    @pl.when((ngrp & 1) != 0)
    def _tail():
        g = ngrp - 1
        wait_grp(g, rows_s0, rows_d0, sem_a0, sem_b0)
        compute_grp(g, rows_s0, rows_d0)

    pltpu.sync_copy(scores_v, out_hbm.at[pl.ds(base_eid, E_PER_TILE)])


def _tc_loss_body(scores_ref, out_ref):
    s = scores_ref[...]
    terms = -jnp.log(jax.nn.sigmoid(s) + EPS)
    out_ref[0, 0] = jnp.sum(terms)


_tc_loss = pl.pallas_call(
    _tc_loss_body,
    out_shape=jax.ShapeDtypeStruct((1, 1), jnp.float32),
    out_specs=pl.BlockSpec(memory_space=pltpu.SMEM),
)


def kernel(A_star, edge_index, node_mask):
    ei = edge_index.astype(jnp.int32)
    src = jnp.pad(ei[0], (0, E_PAD - N_EDGES))
    dst = jnp.pad(ei[1], (0, E_PAD - N_EDGES))
    mask_i = node_mask.astype(jnp.int32)
    table = _scatter_ids(src, dst, mask_i)
    scores = _gather_dot(src, dst, mask_i, A_star, table)
    return _tc_loss(scores.reshape(ROWS_TOTAL, 128))[0, 0]
